# baseline (speedup 1.0000x reference)
import functools

import jax
import jax.numpy as jnp
from jax import lax
from jax.experimental import pallas as pl
from jax.experimental.pallas import tpu as pltpu
from jax.experimental.pallas import tpu_sc as plsc

M = 100000
E = 128
K = 33
KP = 64
B = 5000
NB = M // B
DELTA_PULL = 0.5
DELTA_PUSH = 1.5
ALPHA = 1.0
BETA = 1.0
GAMMA = 0.001

NC = 2
NS = 16
NW = NC * NS
CH = 80
NCH = M // CH
FULL_ROUNDS = NCH // NW
REM = NCH - FULL_ROUNDS * NW


def _sc_segment_sums(embeddings, labels_i32, zsums, zcnts, ones_cnt):
    mesh = plsc.VectorSubcoreMesh(
        core_axis_name="c", subcore_axis_name="s",
        num_cores=NC, num_subcores=NS)

    @functools.partial(
        pl.kernel,
        out_type=[
            jax.ShapeDtypeStruct((NC, KP, E), jnp.float32),
            jax.ShapeDtypeStruct((NC, KP, E), jnp.float32),
        ],
        mesh=mesh,
        scratch_types=[
            pltpu.VMEM((CH, E), jnp.float32),
            pltpu.VMEM((CH,), jnp.int32),
            pltpu.VMEM((CH, E), jnp.float32),
            pltpu.VMEM_SHARED((KP, E), jnp.float32),
            pltpu.VMEM_SHARED((KP, E), jnp.float32),
            pltpu.SemaphoreType.DMA,
            pltpu.SemaphoreType.DMA,
        ])
    def k(emb_hbm, lab_hbm, zs_hbm, zc_hbm, ones_hbm, sums_out, cnts_out,
          buf, idx, onesv, tbl, ctbl, sem1, sem2):
        c = lax.axis_index("c")
        s = lax.axis_index("s")
        gw = c * NS + s

        pltpu.sync_copy(ones_hbm, onesv)

        @pl.when(s == 0)
        def _zero():
            pltpu.sync_copy(zs_hbm, tbl)
            pltpu.sync_copy(zc_hbm, ctbl)

        plsc.subcore_barrier()

        def step(cid):
            base = cid * CH
            pltpu.sync_copy(emb_hbm.at[pl.ds(base, CH)], buf)
            pltpu.sync_copy(lab_hbm.at[pl.ds(base, CH)], idx)
            pltpu.async_copy(buf, tbl.at[idx], sem1, add=True).wait()
            pltpu.async_copy(onesv, ctbl.at[idx], sem2, add=True).wait()

        def body(k_it, carry):
            step(gw + NW * k_it)
            return carry

        lax.fori_loop(0, FULL_ROUNDS, body, 0)

        @pl.when(gw < REM)
        def _tail():
            step(gw + NW * FULL_ROUNDS)

        plsc.subcore_barrier()

        @pl.when(s == 0)
        def _flush():
            pltpu.sync_copy(tbl, sums_out.at[c])
            pltpu.sync_copy(ctbl, cnts_out.at[c])

    return k(embeddings, labels_i32, zsums, zcnts, ones_cnt)


def _tc_body(labc_ref, emb_ref, sums_ref, cnts_ref, out_ref,
             meansT_ref, msqh_ref, msq_ref, counts_ref, pulls_ref):
    g = pl.program_id(0)

    @pl.when(g == 0)
    def _init():
        sums = sums_ref[0] + sums_ref[1]
        counts = cnts_ref[0, :, 0:1] + cnts_ref[1, :, 0:1]
        counts_ref[...] = counts
        safe = jnp.maximum(counts, 1.0)
        means = sums / safe
        meansT = jnp.swapaxes(means, 0, 1)
        meansT_ref[...] = meansT
        msq_ref[...] = jnp.sum(means * means, axis=1, keepdims=True)
        msqh_ref[...] = 0.5 * jnp.sum(meansT * meansT, axis=0, keepdims=True)
        pulls_ref[...] = jnp.zeros_like(pulls_ref)

    lab_c = labc_ref[0]
    iota_row = jax.lax.broadcasted_iota(jnp.int32, (1, KP), 1)
    onehot_bk = (lab_c == iota_row).astype(jnp.float32)
    emb = emb_ref[...]
    e2 = jnp.sum(emb * emb, axis=1, keepdims=True)
    dots = jax.lax.dot_general(
        emb, meansT_ref[...], (((1,), (0,)), ((), ())),
        preferred_element_type=jnp.float32)
    sel = jnp.sum((dots - msqh_ref[...]) * onehot_bk, axis=1,
                  keepdims=True)
    d2 = jnp.maximum(e2 - 2.0 * sel, 0.0)
    dist = jnp.sqrt(d2 + 1e-12)
    pull_b = jnp.square(jnp.maximum(dist - DELTA_PULL, 0.0))
    pulls_ref[...] += jax.lax.dot_general(
        jnp.ones((1, B), jnp.float32), onehot_bk * pull_b,
        (((1,), (0,)), ((), ())),
        preferred_element_type=jnp.float32)

    @pl.when(g == NB - 1)
    def _final():
        counts = counts_ref[...]
        safe = jnp.maximum(counts, 1.0)
        iota_c = jax.lax.broadcasted_iota(jnp.int32, (KP, 1), 0)
        valid = (counts > 0.0) & (iota_c > 0)
        validf = valid.astype(jnp.float32)
        C = jnp.sum(validf)
        Cs = jnp.maximum(C, 1.0)

        ii = jax.lax.broadcasted_iota(jnp.int32, (KP, KP), 0)
        jj = jax.lax.broadcasted_iota(jnp.int32, (KP, KP), 1)
        eye = (ii == jj).astype(jnp.float32)
        safe_row = jnp.sum(eye * safe, axis=0, keepdims=True)
        valid_rowf = jnp.sum(eye * validf, axis=0, keepdims=True)

        pull_loss = jnp.sum(
            jnp.where(valid_rowf > 0.0, pulls_ref[...] / safe_row, 0.0)) / Cs

        meansT = meansT_ref[...]
        means = jnp.swapaxes(meansT, 0, 1)
        msq_col = msq_ref[...]
        msq_row = 2.0 * msqh_ref[...]
        G = jax.lax.dot_general(
            means, meansT, (((1,), (0,)), ((), ())),
            preferred_element_type=jnp.float32)
        sq = jnp.maximum(msq_col + msq_row - 2.0 * G, 0.0)
        pm = validf * valid_rowf * (ii < jj).astype(jnp.float32)
        d = jnp.sqrt(jnp.where(pm > 0.0, sq, 1.0))
        push = jnp.square(jnp.maximum(2.0 * DELTA_PUSH - d, 0.0))
        n_pairs = jnp.sum(pm)
        push_loss = jnp.where(
            n_pairs > 0.0, jnp.sum(push * pm) / jnp.maximum(n_pairs, 1.0), 0.0)

        mnorm = jnp.sqrt(jnp.where(valid, msq_col, 1.0))
        reg_loss = jnp.sum(jnp.where(valid, mnorm, 0.0)) / Cs

        total = ALPHA * pull_loss + BETA * push_loss + GAMMA * reg_loss
        out_ref[...] = jnp.broadcast_to(total, (1, 1))


@jax.jit
def kernel(embeddings, instance_labels):
    labi = instance_labels.astype(jnp.int32)
    zsums = jnp.zeros((KP, E), jnp.float32)
    zcnts = jnp.zeros((KP, E), jnp.float32)
    ones_cnt = jnp.ones((CH, E), jnp.float32)
    sums2, cnts2 = _sc_segment_sums(embeddings, labi, zsums, zcnts, ones_cnt)

    lab_col = labi.reshape(NB, B, 1)
    out = pl.pallas_call(
        _tc_body,
        grid=(NB,),
        in_specs=[
            pl.BlockSpec((1, B, 1), lambda g: (g, 0, 0)),
            pl.BlockSpec((B, E), lambda g: (g, 0)),
            pl.BlockSpec((NC, KP, E), lambda g: (0, 0, 0)),
            pl.BlockSpec((NC, KP, E), lambda g: (0, 0, 0)),
        ],
        out_specs=pl.BlockSpec((1, 1), lambda g: (0, 0)),
        out_shape=jax.ShapeDtypeStruct((1, 1), jnp.float32),
        scratch_shapes=[
            pltpu.VMEM((E, KP), jnp.float32),
            pltpu.VMEM((1, KP), jnp.float32),
            pltpu.VMEM((KP, 1), jnp.float32),
            pltpu.VMEM((KP, 1), jnp.float32),
            pltpu.VMEM((1, KP), jnp.float32),
        ],
    )(lab_col, embeddings, sums2, cnts2)
    return out.reshape(())


# speedup vs baseline: 1.8188x; 1.8188x over previous
import functools

import jax
import jax.numpy as jnp
from jax import lax
from jax.experimental import pallas as pl
from jax.experimental.pallas import tpu as pltpu
from jax.experimental.pallas import tpu_sc as plsc

M = 100000
E = 128
K = 33
KP = 64
B = 5000
NB = M // B
DELTA_PULL = 0.5
DELTA_PUSH = 1.5
ALPHA = 1.0
BETA = 1.0
GAMMA = 0.001

NC = 2
NS = 16
NW = NC * NS
CH = 128
NCHF = M // CH
TAIL = M - NCHF * CH
ROUNDS = (NCHF + NW - 1) // NW

def _sc_segment_sums(embeddings, labels_i32, zsums, zcnts, ones_cnt):
    mesh = plsc.VectorSubcoreMesh(
        core_axis_name="c", subcore_axis_name="s",
        num_cores=NC, num_subcores=NS)

    @functools.partial(
        pl.kernel,
        out_type=[
            jax.ShapeDtypeStruct((NC, KP, E), jnp.float32),
            jax.ShapeDtypeStruct((NC, KP, E), jnp.float32),
        ],
        mesh=mesh,
        scratch_types=[
            pltpu.VMEM((CH, E), jnp.float32),
            pltpu.VMEM((CH, E), jnp.float32),
            pltpu.VMEM((CH,), jnp.int32),
            pltpu.VMEM((CH,), jnp.int32),
            pltpu.VMEM((TAIL, E), jnp.float32),
            pltpu.VMEM((TAIL,), jnp.int32),
            pltpu.VMEM((CH, E), jnp.float32),
            pltpu.VMEM_SHARED((KP, E), jnp.float32),
            pltpu.VMEM_SHARED((KP, E), jnp.float32),
            pltpu.SemaphoreType.DMA,
            pltpu.SemaphoreType.DMA,
            pltpu.SemaphoreType.DMA,
            pltpu.SemaphoreType.DMA,
        ])
    def k(emb_hbm, lab_hbm, zs_hbm, zc_hbm, ones_hbm, sums_out, cnts_out,
          buf0, buf1, idx0, idx1, buft, idxt, onesv, tbl, ctbl,
          sem_st0, sem_st1, sem_sc0, sem_sc1):
        c = lax.axis_index("c")
        s = lax.axis_index("s")
        gw = c * NS + s

        bufs = (buf0, buf1)
        idxs = (idx0, idx1)
        sem_st = (sem_st0, sem_st1)
        sem_sc = (sem_sc0, sem_sc1)

        pltpu.sync_copy(ones_hbm, onesv)

        @pl.when(s == 0)
        def _zero():
            pltpu.sync_copy(zs_hbm, tbl)
            pltpu.sync_copy(zc_hbm, ctbl)

        plsc.subcore_barrier()

        def cid_of(k_it):
            return gw + NW * k_it

        def stage(k_it, b):
            base = cid_of(k_it) * CH
            pltpu.async_copy(emb_hbm.at[pl.ds(base, CH)], bufs[b], sem_st[b])
            pltpu.async_copy(lab_hbm.at[pl.ds(base, CH)], idxs[b], sem_st[b])

        @pl.when(cid_of(0) < NCHF)
        def _prime():
            stage(0, 0)

        for k_it in range(ROUNDS):
            b = k_it % 2

            @pl.when(cid_of(k_it) < NCHF)
            def _iter(k_it=k_it, b=b):
                pltpu.make_async_copy(
                    emb_hbm.at[pl.ds(0, CH)], bufs[b], sem_st[b]).wait()
                pltpu.make_async_copy(
                    lab_hbm.at[pl.ds(0, CH)], idxs[b], sem_st[b]).wait()
                if k_it + 1 < ROUNDS:
                    @pl.when(cid_of(k_it + 1) < NCHF)
                    def _next():
                        stage(k_it + 1, 1 - b)
                pltpu.async_copy(bufs[b], tbl.at[idxs[b]], sem_sc[b],
                                 add=True)
                pltpu.async_copy(onesv, ctbl.at[idxs[b]], sem_sc[b],
                                 add=True)
                pltpu.make_async_copy(bufs[b], tbl.at[idxs[b]],
                                      sem_sc[b]).wait()
                pltpu.make_async_copy(onesv, ctbl.at[idxs[b]],
                                      sem_sc[b]).wait()

        @pl.when(gw == NW - 1)
        def _tail():
            base = NCHF * CH
            pltpu.sync_copy(emb_hbm.at[pl.ds(base, TAIL)], buft)
            pltpu.sync_copy(lab_hbm.at[pl.ds(base, TAIL)], idxt)
            pltpu.async_copy(buft, tbl.at[idxt], sem_sc0, add=True).wait()
            pltpu.async_copy(onesv.at[pl.ds(0, TAIL)], ctbl.at[idxt],
                             sem_sc1, add=True).wait()

        plsc.subcore_barrier()

        @pl.when(s == 0)
        def _flush():
            pltpu.sync_copy(tbl, sums_out.at[c])
            pltpu.sync_copy(ctbl, cnts_out.at[c])

    return k(embeddings, labels_i32, zsums, zcnts, ones_cnt)


def _tc_body(labr_ref, emb_ref, sums_ref, cnts_ref, out_ref,
             means_ref, msqh_ref, msq_ref, counts_ref, pulls_ref):
    g = pl.program_id(0)

    @pl.when(g == 0)
    def _init():
        sums = sums_ref[0] + sums_ref[1]
        counts = cnts_ref[0, :, 0:1] + cnts_ref[1, :, 0:1]
        counts_ref[...] = counts
        safe = jnp.maximum(counts, 1.0)
        means = sums / safe
        means_ref[...] = means
        msq = jnp.sum(means * means, axis=1, keepdims=True)
        msq_ref[...] = msq
        msqh_ref[...] = 0.5 * msq
        pulls_ref[...] = jnp.zeros_like(pulls_ref)

    lab_r = labr_ref[0]
    iota_col = jax.lax.broadcasted_iota(jnp.int32, (KP, 1), 0)
    onehot_kb = (iota_col == lab_r).astype(jnp.float32)
    emb = emb_ref[...]
    dotsT = jax.lax.dot_general(
        means_ref[...], emb, (((1,), (1,)), ((), ())),
        preferred_element_type=jnp.float32)
    e2 = jax.lax.dot_general(
        jnp.ones((1, E), jnp.float32), emb * emb, (((1,), (1,)), ((), ())),
        preferred_element_type=jnp.float32)
    sel = jnp.sum((dotsT - msqh_ref[...]) * onehot_kb, axis=0,
                  keepdims=True)
    d2 = jnp.maximum(e2 - 2.0 * sel, 0.0)
    dist = jnp.sqrt(d2 + 1e-12)
    pull_r = jnp.square(jnp.maximum(dist - DELTA_PULL, 0.0))
    pulls_ref[...] += jnp.sum(onehot_kb * pull_r, axis=1, keepdims=True)

    @pl.when(g == NB - 1)
    def _final():
        counts = counts_ref[...]
        safe = jnp.maximum(counts, 1.0)
        iota_c = jax.lax.broadcasted_iota(jnp.int32, (KP, 1), 0)
        valid = (counts > 0.0) & (iota_c > 0)
        validf = valid.astype(jnp.float32)
        C = jnp.sum(validf)
        Cs = jnp.maximum(C, 1.0)

        ii = jax.lax.broadcasted_iota(jnp.int32, (KP, KP), 0)
        jj = jax.lax.broadcasted_iota(jnp.int32, (KP, KP), 1)
        eye = (ii == jj).astype(jnp.float32)
        safe_row = jnp.sum(eye * safe, axis=0, keepdims=True)
        valid_rowf = jnp.sum(eye * validf, axis=0, keepdims=True)
        msq_col = msq_ref[...]
        msq_row = jnp.sum(eye * msq_col, axis=0, keepdims=True)

        pull_loss = jnp.sum(
            jnp.where(valid, pulls_ref[...] / safe, 0.0)) / Cs

        means = means_ref[...]
        G = jax.lax.dot_general(
            means, means, (((1,), (1,)), ((), ())),
            preferred_element_type=jnp.float32)
        sq = jnp.maximum(msq_col + msq_row - 2.0 * G, 0.0)
        pm = validf * valid_rowf * (ii < jj).astype(jnp.float32)
        d = jnp.sqrt(jnp.where(pm > 0.0, sq, 1.0))
        push = jnp.square(jnp.maximum(2.0 * DELTA_PUSH - d, 0.0))
        n_pairs = jnp.sum(pm)
        push_loss = jnp.where(
            n_pairs > 0.0, jnp.sum(push * pm) / jnp.maximum(n_pairs, 1.0), 0.0)

        mnorm = jnp.sqrt(jnp.where(valid, msq_col, 1.0))
        reg_loss = jnp.sum(jnp.where(valid, mnorm, 0.0)) / Cs

        total = ALPHA * pull_loss + BETA * push_loss + GAMMA * reg_loss
        out_ref[...] = jnp.broadcast_to(total, (1, 1))


@jax.jit
def kernel(embeddings, instance_labels):
    labi = instance_labels.astype(jnp.int32)
    zsums = jnp.zeros((KP, E), jnp.float32)
    zcnts = jnp.zeros((KP, E), jnp.float32)
    ones_cnt = jnp.ones((CH, E), jnp.float32)
    sums2, cnts2 = _sc_segment_sums(embeddings, labi, zsums, zcnts, ones_cnt)

    lab_row = labi.reshape(NB, 1, B)
    out = pl.pallas_call(
        _tc_body,
        grid=(NB,),
        in_specs=[
            pl.BlockSpec((1, 1, B), lambda g: (g, 0, 0)),
            pl.BlockSpec((B, E), lambda g: (g, 0)),
            pl.BlockSpec((NC, KP, E), lambda g: (0, 0, 0)),
            pl.BlockSpec((NC, KP, E), lambda g: (0, 0, 0)),
        ],
        out_specs=pl.BlockSpec((1, 1), lambda g: (0, 0)),
        out_shape=jax.ShapeDtypeStruct((1, 1), jnp.float32),
        scratch_shapes=[
            pltpu.VMEM((KP, E), jnp.float32),
            pltpu.VMEM((KP, 1), jnp.float32),
            pltpu.VMEM((KP, 1), jnp.float32),
            pltpu.VMEM((KP, 1), jnp.float32),
            pltpu.VMEM((KP, 1), jnp.float32),
        ],
    )(lab_row, embeddings, sums2, cnts2)
    return out.reshape(())


# speedup vs baseline: 1.9210x; 1.0562x over previous
import functools

import jax
import jax.numpy as jnp
from jax import lax
from jax.experimental import pallas as pl
from jax.experimental.pallas import tpu as pltpu
from jax.experimental.pallas import tpu_sc as plsc

M = 100000
E = 128
K = 33
KP = 64
B = 10000
NB = M // B
DELTA_PULL = 0.5
DELTA_PUSH = 1.5
ALPHA = 1.0
BETA = 1.0
GAMMA = 0.001

NC = 2
NS = 16
NW = NC * NS
CH = 128
NCHF = M // CH
TAIL = M - NCHF * CH
ROUNDS = (NCHF + NW - 1) // NW

def _sc_segment_sums(embeddings, labels_i32, zsums, zcnts, ones_cnt):
    mesh = plsc.VectorSubcoreMesh(
        core_axis_name="c", subcore_axis_name="s",
        num_cores=NC, num_subcores=NS)

    @functools.partial(
        pl.kernel,
        out_type=[
            jax.ShapeDtypeStruct((NC, KP, E), jnp.float32),
            jax.ShapeDtypeStruct((NC, KP, E), jnp.float32),
        ],
        mesh=mesh,
        scratch_types=[
            pltpu.VMEM((CH, E), jnp.float32),
            pltpu.VMEM((CH, E), jnp.float32),
            pltpu.VMEM((CH,), jnp.int32),
            pltpu.VMEM((CH,), jnp.int32),
            pltpu.VMEM((TAIL, E), jnp.float32),
            pltpu.VMEM((TAIL,), jnp.int32),
            pltpu.VMEM((CH, E), jnp.float32),
            pltpu.VMEM_SHARED((KP, E), jnp.float32),
            pltpu.VMEM_SHARED((KP, E), jnp.float32),
            pltpu.SemaphoreType.DMA,
            pltpu.SemaphoreType.DMA,
            pltpu.SemaphoreType.DMA,
            pltpu.SemaphoreType.DMA,
        ])
    def k(emb_hbm, lab_hbm, zs_hbm, zc_hbm, ones_hbm, sums_out, cnts_out,
          buf0, buf1, idx0, idx1, buft, idxt, onesv, tbl, ctbl,
          sem_st0, sem_st1, sem_sc0, sem_sc1):
        c = lax.axis_index("c")
        s = lax.axis_index("s")
        gw = c * NS + s

        bufs = (buf0, buf1)
        idxs = (idx0, idx1)
        sem_st = (sem_st0, sem_st1)
        sem_sc = (sem_sc0, sem_sc1)

        pltpu.sync_copy(ones_hbm, onesv)

        @pl.when(s == 0)
        def _zero():
            pltpu.sync_copy(zs_hbm, tbl)
            pltpu.sync_copy(zc_hbm, ctbl)

        plsc.subcore_barrier()

        def cid_of(k_it):
            return gw + NW * k_it

        def stage(k_it, b):
            base = cid_of(k_it) * CH
            pltpu.async_copy(emb_hbm.at[pl.ds(base, CH)], bufs[b], sem_st[b])
            pltpu.async_copy(lab_hbm.at[pl.ds(base, CH)], idxs[b], sem_st[b])

        def wait_stage(b):
            pltpu.make_async_copy(
                emb_hbm.at[pl.ds(0, CH)], bufs[b], sem_st[b]).wait()
            pltpu.make_async_copy(
                lab_hbm.at[pl.ds(0, CH)], idxs[b], sem_st[b]).wait()

        def fire_scatter(b):
            pltpu.async_copy(bufs[b], tbl.at[idxs[b]], sem_sc[b], add=True)
            pltpu.async_copy(onesv, ctbl.at[idxs[b]], sem_sc[b], add=True)

        def drain_scatter(b):
            pltpu.make_async_copy(bufs[b], tbl.at[idxs[b]], sem_sc[b]).wait()
            pltpu.make_async_copy(onesv, ctbl.at[idxs[b]], sem_sc[b]).wait()

        @pl.when(cid_of(0) < NCHF)
        def _prime():
            stage(0, 0)

        for k_it in range(ROUNDS):
            b = k_it % 2
            if k_it >= 1:
                @pl.when(cid_of(k_it - 1) < NCHF)
                def _drain_prev(b=b):
                    drain_scatter(1 - b)

            @pl.when(cid_of(k_it) < NCHF)
            def _iter(k_it=k_it, b=b):
                if k_it + 1 < ROUNDS:
                    @pl.when(cid_of(k_it + 1) < NCHF)
                    def _next():
                        stage(k_it + 1, 1 - b)
                wait_stage(b)
                fire_scatter(b)

        @pl.when(cid_of(ROUNDS - 1) < NCHF)
        def _drain_last():
            drain_scatter((ROUNDS - 1) % 2)

        @pl.when(gw == NW - 1)
        def _tail():
            base = NCHF * CH
            pltpu.sync_copy(emb_hbm.at[pl.ds(base, TAIL)], buft)
            pltpu.sync_copy(lab_hbm.at[pl.ds(base, TAIL)], idxt)
            pltpu.async_copy(buft, tbl.at[idxt], sem_sc0, add=True).wait()
            pltpu.async_copy(onesv.at[pl.ds(0, TAIL)], ctbl.at[idxt],
                             sem_sc1, add=True).wait()

        plsc.subcore_barrier()

        @pl.when(s == 0)
        def _flush():
            pltpu.sync_copy(tbl, sums_out.at[c])
            pltpu.sync_copy(ctbl, cnts_out.at[c])

    return k(embeddings, labels_i32, zsums, zcnts, ones_cnt)


def _tc_body(labr_ref, emb_ref, sums_ref, cnts_ref, out_ref,
             means_ref, msqh_ref, msq_ref, counts_ref, pulls_ref):
    g = pl.program_id(0)

    @pl.when(g == 0)
    def _init():
        sums = sums_ref[0] + sums_ref[1]
        counts = cnts_ref[0, :, 0:1] + cnts_ref[1, :, 0:1]
        counts_ref[...] = counts
        safe = jnp.maximum(counts, 1.0)
        means = sums / safe
        means_ref[...] = means
        msq = jnp.sum(means * means, axis=1, keepdims=True)
        msq_ref[...] = msq
        msqh_ref[...] = 0.5 * msq
        pulls_ref[...] = jnp.zeros_like(pulls_ref)

    lab_r = labr_ref[0]
    iota_col = jax.lax.broadcasted_iota(jnp.int32, (KP, 1), 0)
    onehot_kb = (iota_col == lab_r).astype(jnp.float32)
    emb = emb_ref[...]
    dotsT = jax.lax.dot_general(
        means_ref[...], emb, (((1,), (1,)), ((), ())),
        preferred_element_type=jnp.float32)
    e2 = jax.lax.dot_general(
        jnp.ones((1, E), jnp.float32), emb * emb, (((1,), (1,)), ((), ())),
        preferred_element_type=jnp.float32)
    sel = jnp.sum((dotsT - msqh_ref[...]) * onehot_kb, axis=0,
                  keepdims=True)
    d2 = jnp.maximum(e2 - 2.0 * sel, 0.0)
    dist = jnp.sqrt(d2 + 1e-12)
    pull_r = jnp.square(jnp.maximum(dist - DELTA_PULL, 0.0))
    pulls_ref[...] += jnp.sum(onehot_kb * pull_r, axis=1, keepdims=True)

    @pl.when(g == NB - 1)
    def _final():
        counts = counts_ref[...]
        safe = jnp.maximum(counts, 1.0)
        iota_c = jax.lax.broadcasted_iota(jnp.int32, (KP, 1), 0)
        valid = (counts > 0.0) & (iota_c > 0)
        validf = valid.astype(jnp.float32)
        C = jnp.sum(validf)
        Cs = jnp.maximum(C, 1.0)

        ii = jax.lax.broadcasted_iota(jnp.int32, (KP, KP), 0)
        jj = jax.lax.broadcasted_iota(jnp.int32, (KP, KP), 1)
        eye = (ii == jj).astype(jnp.float32)
        safe_row = jnp.sum(eye * safe, axis=0, keepdims=True)
        valid_rowf = jnp.sum(eye * validf, axis=0, keepdims=True)
        msq_col = msq_ref[...]
        msq_row = jnp.sum(eye * msq_col, axis=0, keepdims=True)

        pull_loss = jnp.sum(
            jnp.where(valid, pulls_ref[...] / safe, 0.0)) / Cs

        means = means_ref[...]
        G = jax.lax.dot_general(
            means, means, (((1,), (1,)), ((), ())),
            preferred_element_type=jnp.float32)
        sq = jnp.maximum(msq_col + msq_row - 2.0 * G, 0.0)
        pm = validf * valid_rowf * (ii < jj).astype(jnp.float32)
        d = jnp.sqrt(jnp.where(pm > 0.0, sq, 1.0))
        push = jnp.square(jnp.maximum(2.0 * DELTA_PUSH - d, 0.0))
        n_pairs = jnp.sum(pm)
        push_loss = jnp.where(
            n_pairs > 0.0, jnp.sum(push * pm) / jnp.maximum(n_pairs, 1.0), 0.0)

        mnorm = jnp.sqrt(jnp.where(valid, msq_col, 1.0))
        reg_loss = jnp.sum(jnp.where(valid, mnorm, 0.0)) / Cs

        total = ALPHA * pull_loss + BETA * push_loss + GAMMA * reg_loss
        out_ref[...] = jnp.broadcast_to(total, (1, 1))


@jax.jit
def kernel(embeddings, instance_labels):
    labi = instance_labels.astype(jnp.int32)
    zsums = jnp.zeros((KP, E), jnp.float32)
    zcnts = jnp.zeros((KP, E), jnp.float32)
    ones_cnt = jnp.ones((CH, E), jnp.float32)
    sums2, cnts2 = _sc_segment_sums(embeddings, labi, zsums, zcnts, ones_cnt)

    lab_row = labi.reshape(NB, 1, B)
    out = pl.pallas_call(
        _tc_body,
        grid=(NB,),
        in_specs=[
            pl.BlockSpec((1, 1, B), lambda g: (g, 0, 0)),
            pl.BlockSpec((B, E), lambda g: (g, 0)),
            pl.BlockSpec((NC, KP, E), lambda g: (0, 0, 0)),
            pl.BlockSpec((NC, KP, E), lambda g: (0, 0, 0)),
        ],
        out_specs=pl.BlockSpec((1, 1), lambda g: (0, 0)),
        out_shape=jax.ShapeDtypeStruct((1, 1), jnp.float32),
        scratch_shapes=[
            pltpu.VMEM((KP, E), jnp.float32),
            pltpu.VMEM((KP, 1), jnp.float32),
            pltpu.VMEM((KP, 1), jnp.float32),
            pltpu.VMEM((KP, 1), jnp.float32),
            pltpu.VMEM((KP, 1), jnp.float32),
        ],
    )(lab_row, embeddings, sums2, cnts2)
    return out.reshape(())


# speedup vs baseline: 2.3824x; 1.2402x over previous
import functools

import jax
import jax.numpy as jnp
from jax import lax
from jax.experimental import pallas as pl
from jax.experimental.pallas import tpu as pltpu
from jax.experimental.pallas import tpu_sc as plsc

M = 100000
E = 128
K = 33
KP = 64
B = 10000
NB = M // B
DELTA_PULL = 0.5
DELTA_PUSH = 1.5
ALPHA = 1.0
BETA = 1.0
GAMMA = 0.001

NC = 2
NS = 16
NW = NC * NS
RTC = 50000
RSC = M - RTC
CH = 128
NCHF = RSC // CH
TAIL = RSC - NCHF * CH
ROUNDS = (NCHF + NW - 1) // NW
B1 = 10000
NB1 = RTC // B1

def _sc_segment_sums(embeddings, labels_i32, zsums, zcnts, ones_cnt):
    mesh = plsc.VectorSubcoreMesh(
        core_axis_name="c", subcore_axis_name="s",
        num_cores=NC, num_subcores=NS)

    @functools.partial(
        pl.kernel,
        out_type=[
            jax.ShapeDtypeStruct((NC, KP, E), jnp.float32),
            jax.ShapeDtypeStruct((NC, KP, E), jnp.float32),
        ],
        mesh=mesh,
        scratch_types=[
            pltpu.VMEM((CH, E), jnp.float32),
            pltpu.VMEM((CH, E), jnp.float32),
            pltpu.VMEM((CH,), jnp.int32),
            pltpu.VMEM((CH,), jnp.int32),
            pltpu.VMEM((TAIL, E), jnp.float32),
            pltpu.VMEM((TAIL,), jnp.int32),
            pltpu.VMEM((CH, E), jnp.float32),
            pltpu.VMEM_SHARED((KP, E), jnp.float32),
            pltpu.VMEM_SHARED((KP, E), jnp.float32),
            pltpu.SemaphoreType.DMA,
            pltpu.SemaphoreType.DMA,
            pltpu.SemaphoreType.DMA,
            pltpu.SemaphoreType.DMA,
        ])
    def k(emb_hbm, lab_hbm, zs_hbm, zc_hbm, ones_hbm, sums_out, cnts_out,
          buf0, buf1, idx0, idx1, buft, idxt, onesv, tbl, ctbl,
          sem_st0, sem_st1, sem_sc0, sem_sc1):
        c = lax.axis_index("c")
        s = lax.axis_index("s")
        gw = c * NS + s

        bufs = (buf0, buf1)
        idxs = (idx0, idx1)
        sem_st = (sem_st0, sem_st1)
        sem_sc = (sem_sc0, sem_sc1)

        pltpu.sync_copy(ones_hbm, onesv)

        @pl.when(s == 0)
        def _zero():
            pltpu.sync_copy(zs_hbm, tbl)
            pltpu.sync_copy(zc_hbm, ctbl)

        plsc.subcore_barrier()

        def cid_of(k_it):
            return gw + NW * k_it

        def stage(k_it, b):
            base = RTC + cid_of(k_it) * CH
            pltpu.async_copy(emb_hbm.at[pl.ds(base, CH)], bufs[b], sem_st[b])
            pltpu.async_copy(lab_hbm.at[pl.ds(base, CH)], idxs[b], sem_st[b])

        def wait_stage(b):
            pltpu.make_async_copy(
                emb_hbm.at[pl.ds(0, CH)], bufs[b], sem_st[b]).wait()
            pltpu.make_async_copy(
                lab_hbm.at[pl.ds(0, CH)], idxs[b], sem_st[b]).wait()

        def fire_scatter(b):
            pltpu.async_copy(bufs[b], tbl.at[idxs[b]], sem_sc[b], add=True)
            pltpu.async_copy(onesv, ctbl.at[idxs[b]], sem_sc[b], add=True)

        def drain_scatter(b):
            pltpu.make_async_copy(bufs[b], tbl.at[idxs[b]], sem_sc[b]).wait()
            pltpu.make_async_copy(onesv, ctbl.at[idxs[b]], sem_sc[b]).wait()

        @pl.when(cid_of(0) < NCHF)
        def _prime():
            stage(0, 0)

        for k_it in range(ROUNDS):
            b = k_it % 2
            if k_it >= 1:
                @pl.when(cid_of(k_it - 1) < NCHF)
                def _drain_prev(b=b):
                    drain_scatter(1 - b)

            @pl.when(cid_of(k_it) < NCHF)
            def _iter(k_it=k_it, b=b):
                if k_it + 1 < ROUNDS:
                    @pl.when(cid_of(k_it + 1) < NCHF)
                    def _next():
                        stage(k_it + 1, 1 - b)
                wait_stage(b)
                fire_scatter(b)

        @pl.when(cid_of(ROUNDS - 1) < NCHF)
        def _drain_last():
            drain_scatter((ROUNDS - 1) % 2)

        @pl.when(gw == NW - 1)
        def _tail():
            base = RTC + NCHF * CH
            pltpu.sync_copy(emb_hbm.at[pl.ds(base, TAIL)], buft)
            pltpu.sync_copy(lab_hbm.at[pl.ds(base, TAIL)], idxt)
            pltpu.async_copy(buft, tbl.at[idxt], sem_sc0, add=True).wait()
            pltpu.async_copy(onesv.at[pl.ds(0, TAIL)], ctbl.at[idxt],
                             sem_sc1, add=True).wait()

        plsc.subcore_barrier()

        @pl.when(s == 0)
        def _flush():
            pltpu.sync_copy(tbl, sums_out.at[c])
            pltpu.sync_copy(ctbl, cnts_out.at[c])

    return k(embeddings, labels_i32, zsums, zcnts, ones_cnt)


def _tc_p1_body(labr_ref, emb_ref, sums_ref, counts_ref):
    g = pl.program_id(0)

    @pl.when(g == 0)
    def _init():
        sums_ref[...] = jnp.zeros_like(sums_ref)
        counts_ref[...] = jnp.zeros_like(counts_ref)

    lab_r = labr_ref[0]
    iota_col = jax.lax.broadcasted_iota(jnp.int32, (KP, 1), 0)
    onehot_kb = (iota_col == lab_r).astype(jnp.float32)
    emb = emb_ref[...]
    sums_ref[...] += jax.lax.dot_general(
        onehot_kb, emb, (((1,), (0,)), ((), ())),
        preferred_element_type=jnp.float32)
    counts_ref[...] += jnp.sum(onehot_kb, axis=1, keepdims=True)


def _tc_partial_sums(embeddings, labi):
    lab_row = labi.reshape(M // B1, 1, B1)
    return pl.pallas_call(
        _tc_p1_body,
        grid=(NB1,),
        in_specs=[
            pl.BlockSpec((1, 1, B1), lambda g: (g, 0, 0)),
            pl.BlockSpec((B1, E), lambda g: (g, 0)),
        ],
        out_specs=[
            pl.BlockSpec((KP, E), lambda g: (0, 0)),
            pl.BlockSpec((KP, 1), lambda g: (0, 0)),
        ],
        out_shape=[
            jax.ShapeDtypeStruct((KP, E), jnp.float32),
            jax.ShapeDtypeStruct((KP, 1), jnp.float32),
        ],
    )(lab_row, embeddings)


def _tc_body(labr_ref, emb_ref, sums_ref, cnts_ref, stc_ref, ctc_ref, out_ref,
             means_ref, msqh_ref, msq_ref, counts_ref, pulls_ref):
    g = pl.program_id(0)

    @pl.when(g == 0)
    def _init():
        sums = sums_ref[0] + sums_ref[1] + stc_ref[...]
        counts = (cnts_ref[0, :, 0:1] + cnts_ref[1, :, 0:1]
                  + ctc_ref[...])
        counts_ref[...] = counts
        safe = jnp.maximum(counts, 1.0)
        means = sums / safe
        means_ref[...] = means
        msq = jnp.sum(means * means, axis=1, keepdims=True)
        msq_ref[...] = msq
        msqh_ref[...] = 0.5 * msq
        pulls_ref[...] = jnp.zeros_like(pulls_ref)

    lab_r = labr_ref[0]
    iota_col = jax.lax.broadcasted_iota(jnp.int32, (KP, 1), 0)
    onehot_kb = (iota_col == lab_r).astype(jnp.float32)
    emb = emb_ref[...]
    dotsT = jax.lax.dot_general(
        means_ref[...], emb, (((1,), (1,)), ((), ())),
        preferred_element_type=jnp.float32)
    e2 = jax.lax.dot_general(
        jnp.ones((1, E), jnp.float32), emb * emb, (((1,), (1,)), ((), ())),
        preferred_element_type=jnp.float32)
    sel = jnp.sum((dotsT - msqh_ref[...]) * onehot_kb, axis=0,
                  keepdims=True)
    d2 = jnp.maximum(e2 - 2.0 * sel, 0.0)
    dist = jnp.sqrt(d2 + 1e-12)
    pull_r = jnp.square(jnp.maximum(dist - DELTA_PULL, 0.0))
    pulls_ref[...] += jnp.sum(onehot_kb * pull_r, axis=1, keepdims=True)

    @pl.when(g == NB - 1)
    def _final():
        counts = counts_ref[...]
        safe = jnp.maximum(counts, 1.0)
        iota_c = jax.lax.broadcasted_iota(jnp.int32, (KP, 1), 0)
        valid = (counts > 0.0) & (iota_c > 0)
        validf = valid.astype(jnp.float32)
        C = jnp.sum(validf)
        Cs = jnp.maximum(C, 1.0)

        ii = jax.lax.broadcasted_iota(jnp.int32, (KP, KP), 0)
        jj = jax.lax.broadcasted_iota(jnp.int32, (KP, KP), 1)
        eye = (ii == jj).astype(jnp.float32)
        safe_row = jnp.sum(eye * safe, axis=0, keepdims=True)
        valid_rowf = jnp.sum(eye * validf, axis=0, keepdims=True)
        msq_col = msq_ref[...]
        msq_row = jnp.sum(eye * msq_col, axis=0, keepdims=True)

        pull_loss = jnp.sum(
            jnp.where(valid, pulls_ref[...] / safe, 0.0)) / Cs

        means = means_ref[...]
        G = jax.lax.dot_general(
            means, means, (((1,), (1,)), ((), ())),
            preferred_element_type=jnp.float32)
        sq = jnp.maximum(msq_col + msq_row - 2.0 * G, 0.0)
        pm = validf * valid_rowf * (ii < jj).astype(jnp.float32)
        d = jnp.sqrt(jnp.where(pm > 0.0, sq, 1.0))
        push = jnp.square(jnp.maximum(2.0 * DELTA_PUSH - d, 0.0))
        n_pairs = jnp.sum(pm)
        push_loss = jnp.where(
            n_pairs > 0.0, jnp.sum(push * pm) / jnp.maximum(n_pairs, 1.0), 0.0)

        mnorm = jnp.sqrt(jnp.where(valid, msq_col, 1.0))
        reg_loss = jnp.sum(jnp.where(valid, mnorm, 0.0)) / Cs

        total = ALPHA * pull_loss + BETA * push_loss + GAMMA * reg_loss
        out_ref[...] = jnp.broadcast_to(total, (1, 1))


@jax.jit
def kernel(embeddings, instance_labels):
    labi = instance_labels.astype(jnp.int32)
    zsums = jnp.zeros((KP, E), jnp.float32)
    zcnts = jnp.zeros((KP, E), jnp.float32)
    ones_cnt = jnp.ones((CH, E), jnp.float32)
    sums2, cnts2 = _sc_segment_sums(embeddings, labi, zsums, zcnts, ones_cnt)
    sums_tc, cnts_tc = _tc_partial_sums(embeddings, labi)

    lab_row = labi.reshape(NB, 1, B)
    out = pl.pallas_call(
        _tc_body,
        grid=(NB,),
        in_specs=[
            pl.BlockSpec((1, 1, B), lambda g: (g, 0, 0)),
            pl.BlockSpec((B, E), lambda g: (g, 0)),
            pl.BlockSpec((NC, KP, E), lambda g: (0, 0, 0)),
            pl.BlockSpec((NC, KP, E), lambda g: (0, 0, 0)),
            pl.BlockSpec((KP, E), lambda g: (0, 0)),
            pl.BlockSpec((KP, 1), lambda g: (0, 0)),
        ],
        out_specs=pl.BlockSpec((1, 1), lambda g: (0, 0)),
        out_shape=jax.ShapeDtypeStruct((1, 1), jnp.float32),
        scratch_shapes=[
            pltpu.VMEM((KP, E), jnp.float32),
            pltpu.VMEM((KP, 1), jnp.float32),
            pltpu.VMEM((KP, 1), jnp.float32),
            pltpu.VMEM((KP, 1), jnp.float32),
            pltpu.VMEM((KP, 1), jnp.float32),
        ],
    )(lab_row, embeddings, sums2, cnts2, sums_tc, cnts_tc)
    return out.reshape(())


# speedup vs baseline: 2.6519x; 1.1131x over previous
import functools

import jax
import jax.numpy as jnp
from jax import lax
from jax.experimental import pallas as pl
from jax.experimental.pallas import tpu as pltpu
from jax.experimental.pallas import tpu_sc as plsc

M = 100000
E = 128
K = 33
KP = 64
B = 10000
NB = M // B
DELTA_PULL = 0.5
DELTA_PUSH = 1.5
ALPHA = 1.0
BETA = 1.0
GAMMA = 0.001

NC = 2
NS = 16
NW = NC * NS
RTC = 70000
RSC = M - RTC
CH = 128
NCHF = RSC // CH
TAIL = RSC - NCHF * CH
ROUNDS = (NCHF + NW - 1) // NW
B1 = 10000
NB1 = RTC // B1

def _sc_segment_sums(embeddings, labels_i32, zsums, zcnts, ones_cnt):
    mesh = plsc.VectorSubcoreMesh(
        core_axis_name="c", subcore_axis_name="s",
        num_cores=NC, num_subcores=NS)

    @functools.partial(
        pl.kernel,
        out_type=[
            jax.ShapeDtypeStruct((NC, KP, E), jnp.float32),
            jax.ShapeDtypeStruct((NC, KP, E), jnp.float32),
        ],
        mesh=mesh,
        scratch_types=[
            pltpu.VMEM((CH, E), jnp.float32),
            pltpu.VMEM((CH, E), jnp.float32),
            pltpu.VMEM((CH,), jnp.int32),
            pltpu.VMEM((CH,), jnp.int32),
            pltpu.VMEM((TAIL, E), jnp.float32),
            pltpu.VMEM((TAIL,), jnp.int32),
            pltpu.VMEM((CH, E), jnp.float32),
            pltpu.VMEM_SHARED((KP, E), jnp.float32),
            pltpu.VMEM_SHARED((KP, E), jnp.float32),
            pltpu.SemaphoreType.DMA,
            pltpu.SemaphoreType.DMA,
            pltpu.SemaphoreType.DMA,
            pltpu.SemaphoreType.DMA,
        ])
    def k(emb_hbm, lab_hbm, zs_hbm, zc_hbm, ones_hbm, sums_out, cnts_out,
          buf0, buf1, idx0, idx1, buft, idxt, onesv, tbl, ctbl,
          sem_st0, sem_st1, sem_sc0, sem_sc1):
        c = lax.axis_index("c")
        s = lax.axis_index("s")
        gw = c * NS + s

        bufs = (buf0, buf1)
        idxs = (idx0, idx1)
        sem_st = (sem_st0, sem_st1)
        sem_sc = (sem_sc0, sem_sc1)

        pltpu.sync_copy(ones_hbm, onesv)

        @pl.when(s == 0)
        def _zero():
            pltpu.sync_copy(zs_hbm, tbl)
            pltpu.sync_copy(zc_hbm, ctbl)

        plsc.subcore_barrier()

        def cid_of(k_it):
            return gw + NW * k_it

        def stage(k_it, b):
            base = RTC + cid_of(k_it) * CH
            pltpu.async_copy(emb_hbm.at[pl.ds(base, CH)], bufs[b], sem_st[b])
            pltpu.async_copy(lab_hbm.at[pl.ds(base, CH)], idxs[b], sem_st[b])

        def wait_stage(b):
            pltpu.make_async_copy(
                emb_hbm.at[pl.ds(0, CH)], bufs[b], sem_st[b]).wait()
            pltpu.make_async_copy(
                lab_hbm.at[pl.ds(0, CH)], idxs[b], sem_st[b]).wait()

        def fire_scatter(b):
            pltpu.async_copy(bufs[b], tbl.at[idxs[b]], sem_sc[b], add=True)
            pltpu.async_copy(onesv, ctbl.at[idxs[b]], sem_sc[b], add=True)

        def drain_scatter(b):
            pltpu.make_async_copy(bufs[b], tbl.at[idxs[b]], sem_sc[b]).wait()
            pltpu.make_async_copy(onesv, ctbl.at[idxs[b]], sem_sc[b]).wait()

        @pl.when(cid_of(0) < NCHF)
        def _prime():
            stage(0, 0)

        for k_it in range(ROUNDS):
            b = k_it % 2
            if k_it >= 1:
                @pl.when(cid_of(k_it - 1) < NCHF)
                def _drain_prev(b=b):
                    drain_scatter(1 - b)

            @pl.when(cid_of(k_it) < NCHF)
            def _iter(k_it=k_it, b=b):
                if k_it + 1 < ROUNDS:
                    @pl.when(cid_of(k_it + 1) < NCHF)
                    def _next():
                        stage(k_it + 1, 1 - b)
                wait_stage(b)
                fire_scatter(b)

        @pl.when(cid_of(ROUNDS - 1) < NCHF)
        def _drain_last():
            drain_scatter((ROUNDS - 1) % 2)

        @pl.when(gw == NW - 1)
        def _tail():
            base = RTC + NCHF * CH
            pltpu.sync_copy(emb_hbm.at[pl.ds(base, TAIL)], buft)
            pltpu.sync_copy(lab_hbm.at[pl.ds(base, TAIL)], idxt)
            pltpu.async_copy(buft, tbl.at[idxt], sem_sc0, add=True).wait()
            pltpu.async_copy(onesv.at[pl.ds(0, TAIL)], ctbl.at[idxt],
                             sem_sc1, add=True).wait()

        plsc.subcore_barrier()

        @pl.when(s == 0)
        def _flush():
            pltpu.sync_copy(tbl, sums_out.at[c])
            pltpu.sync_copy(ctbl, cnts_out.at[c])

    return k(embeddings, labels_i32, zsums, zcnts, ones_cnt)


def _tc_p1_body(labr_ref, emb_ref, sums_ref, counts_ref):
    g = pl.program_id(0)

    @pl.when(g == 0)
    def _init():
        sums_ref[...] = jnp.zeros_like(sums_ref)
        counts_ref[...] = jnp.zeros_like(counts_ref)

    lab_r = labr_ref[0]
    iota_col = jax.lax.broadcasted_iota(jnp.int32, (KP, 1), 0)
    onehot_kb = (iota_col == lab_r).astype(jnp.float32)
    emb = emb_ref[...]
    sums_ref[...] += jax.lax.dot_general(
        onehot_kb, emb, (((1,), (0,)), ((), ())),
        preferred_element_type=jnp.float32)
    counts_ref[...] += jnp.sum(onehot_kb, axis=1, keepdims=True)


def _tc_partial_sums(embeddings, labi):
    lab_row = labi.reshape(M // B1, 1, B1)
    return pl.pallas_call(
        _tc_p1_body,
        grid=(NB1,),
        in_specs=[
            pl.BlockSpec((1, 1, B1), lambda g: (g, 0, 0)),
            pl.BlockSpec((B1, E), lambda g: (g, 0)),
        ],
        out_specs=[
            pl.BlockSpec((KP, E), lambda g: (0, 0)),
            pl.BlockSpec((KP, 1), lambda g: (0, 0)),
        ],
        out_shape=[
            jax.ShapeDtypeStruct((KP, E), jnp.float32),
            jax.ShapeDtypeStruct((KP, 1), jnp.float32),
        ],
    )(lab_row, embeddings)


def _tc_body(labr_ref, emb_ref, sums_ref, cnts_ref, stc_ref, ctc_ref, out_ref,
             means_ref, msqh_ref, msq_ref, counts_ref, pulls_ref):
    g = pl.program_id(0)

    @pl.when(g == 0)
    def _init():
        sums = sums_ref[0] + sums_ref[1] + stc_ref[...]
        counts = (cnts_ref[0, :, 0:1] + cnts_ref[1, :, 0:1]
                  + ctc_ref[...])
        counts_ref[...] = counts
        safe = jnp.maximum(counts, 1.0)
        means = sums / safe
        means_ref[...] = means
        msq = jnp.sum(means * means, axis=1, keepdims=True)
        msq_ref[...] = msq
        msqh_ref[...] = 0.5 * msq
        pulls_ref[...] = jnp.zeros_like(pulls_ref)

    lab_r = labr_ref[0]
    iota_col = jax.lax.broadcasted_iota(jnp.int32, (KP, 1), 0)
    onehot_kb = (iota_col == lab_r).astype(jnp.float32)
    emb = emb_ref[...]
    dotsT = jax.lax.dot_general(
        means_ref[...], emb, (((1,), (1,)), ((), ())),
        preferred_element_type=jnp.float32)
    e2 = jax.lax.dot_general(
        jnp.ones((1, E), jnp.float32), emb * emb, (((1,), (1,)), ((), ())),
        preferred_element_type=jnp.float32)
    sel = jnp.sum((dotsT - msqh_ref[...]) * onehot_kb, axis=0,
                  keepdims=True)
    d2 = jnp.maximum(e2 - 2.0 * sel, 0.0)
    dist = jnp.sqrt(d2 + 1e-12)
    pull_r = jnp.square(jnp.maximum(dist - DELTA_PULL, 0.0))
    pulls_ref[...] += jnp.sum(onehot_kb * pull_r, axis=1, keepdims=True)

    @pl.when(g == NB - 1)
    def _final():
        counts = counts_ref[...]
        safe = jnp.maximum(counts, 1.0)
        iota_c = jax.lax.broadcasted_iota(jnp.int32, (KP, 1), 0)
        valid = (counts > 0.0) & (iota_c > 0)
        validf = valid.astype(jnp.float32)
        C = jnp.sum(validf)
        Cs = jnp.maximum(C, 1.0)

        ii = jax.lax.broadcasted_iota(jnp.int32, (KP, KP), 0)
        jj = jax.lax.broadcasted_iota(jnp.int32, (KP, KP), 1)
        eye = (ii == jj).astype(jnp.float32)
        safe_row = jnp.sum(eye * safe, axis=0, keepdims=True)
        valid_rowf = jnp.sum(eye * validf, axis=0, keepdims=True)
        msq_col = msq_ref[...]
        msq_row = jnp.sum(eye * msq_col, axis=0, keepdims=True)

        pull_loss = jnp.sum(
            jnp.where(valid, pulls_ref[...] / safe, 0.0)) / Cs

        means = means_ref[...]
        G = jax.lax.dot_general(
            means, means, (((1,), (1,)), ((), ())),
            preferred_element_type=jnp.float32)
        sq = jnp.maximum(msq_col + msq_row - 2.0 * G, 0.0)
        pm = validf * valid_rowf * (ii < jj).astype(jnp.float32)
        d = jnp.sqrt(jnp.where(pm > 0.0, sq, 1.0))
        push = jnp.square(jnp.maximum(2.0 * DELTA_PUSH - d, 0.0))
        n_pairs = jnp.sum(pm)
        push_loss = jnp.where(
            n_pairs > 0.0, jnp.sum(push * pm) / jnp.maximum(n_pairs, 1.0), 0.0)

        mnorm = jnp.sqrt(jnp.where(valid, msq_col, 1.0))
        reg_loss = jnp.sum(jnp.where(valid, mnorm, 0.0)) / Cs

        total = ALPHA * pull_loss + BETA * push_loss + GAMMA * reg_loss
        out_ref[...] = jnp.broadcast_to(total, (1, 1))


@jax.jit
def kernel(embeddings, instance_labels):
    labi = instance_labels.astype(jnp.int32)
    zsums = jnp.zeros((KP, E), jnp.float32)
    zcnts = jnp.zeros((KP, E), jnp.float32)
    ones_cnt = jnp.ones((CH, E), jnp.float32)
    sums2, cnts2 = _sc_segment_sums(embeddings, labi, zsums, zcnts, ones_cnt)
    sums_tc, cnts_tc = _tc_partial_sums(embeddings, labi)

    lab_row = labi.reshape(NB, 1, B)
    out = pl.pallas_call(
        _tc_body,
        grid=(NB,),
        in_specs=[
            pl.BlockSpec((1, 1, B), lambda g: (g, 0, 0)),
            pl.BlockSpec((B, E), lambda g: (g, 0)),
            pl.BlockSpec((NC, KP, E), lambda g: (0, 0, 0)),
            pl.BlockSpec((NC, KP, E), lambda g: (0, 0, 0)),
            pl.BlockSpec((KP, E), lambda g: (0, 0)),
            pl.BlockSpec((KP, 1), lambda g: (0, 0)),
        ],
        out_specs=pl.BlockSpec((1, 1), lambda g: (0, 0)),
        out_shape=jax.ShapeDtypeStruct((1, 1), jnp.float32),
        scratch_shapes=[
            pltpu.VMEM((KP, E), jnp.float32),
            pltpu.VMEM((KP, 1), jnp.float32),
            pltpu.VMEM((KP, 1), jnp.float32),
            pltpu.VMEM((KP, 1), jnp.float32),
            pltpu.VMEM((KP, 1), jnp.float32),
        ],
    )(lab_row, embeddings, sums2, cnts2, sums_tc, cnts_tc)
    return out.reshape(())


# speedup vs baseline: 2.8925x; 1.0908x over previous
import functools

import jax
import jax.numpy as jnp
from jax import lax
from jax.experimental import pallas as pl
from jax.experimental.pallas import tpu as pltpu
from jax.experimental.pallas import tpu_sc as plsc

M = 100000
E = 128
K = 33
KP = 48
B = 10000
NB = M // B
DELTA_PULL = 0.5
DELTA_PUSH = 1.5
ALPHA = 1.0
BETA = 1.0
GAMMA = 0.001

NC = 2
NS = 16
NW = NC * NS
RTC = 80000
RSC = M - RTC
CH = 128
NCHF = RSC // CH
TAIL = RSC - NCHF * CH
ROUNDS = (NCHF + NW - 1) // NW
B1 = 10000
NB1 = RTC // B1

def _sc_segment_sums(embeddings, labels_i32, zsums, zcnts, ones_cnt):
    mesh = plsc.VectorSubcoreMesh(
        core_axis_name="c", subcore_axis_name="s",
        num_cores=NC, num_subcores=NS)

    @functools.partial(
        pl.kernel,
        out_type=[
            jax.ShapeDtypeStruct((NC, KP, E), jnp.float32),
            jax.ShapeDtypeStruct((NC, KP, E), jnp.float32),
        ],
        mesh=mesh,
        scratch_types=[
            pltpu.VMEM((CH, E), jnp.float32),
            pltpu.VMEM((CH, E), jnp.float32),
            pltpu.VMEM((CH,), jnp.int32),
            pltpu.VMEM((CH,), jnp.int32),
            pltpu.VMEM((TAIL, E), jnp.float32),
            pltpu.VMEM((TAIL,), jnp.int32),
            pltpu.VMEM((CH, E), jnp.float32),
            pltpu.VMEM_SHARED((KP, E), jnp.float32),
            pltpu.VMEM_SHARED((KP, E), jnp.float32),
            pltpu.SemaphoreType.DMA,
            pltpu.SemaphoreType.DMA,
            pltpu.SemaphoreType.DMA,
            pltpu.SemaphoreType.DMA,
        ])
    def k(emb_hbm, lab_hbm, zs_hbm, zc_hbm, ones_hbm, sums_out, cnts_out,
          buf0, buf1, idx0, idx1, buft, idxt, onesv, tbl, ctbl,
          sem_st0, sem_st1, sem_sc0, sem_sc1):
        c = lax.axis_index("c")
        s = lax.axis_index("s")
        gw = c * NS + s

        bufs = (buf0, buf1)
        idxs = (idx0, idx1)
        sem_st = (sem_st0, sem_st1)
        sem_sc = (sem_sc0, sem_sc1)

        pltpu.sync_copy(ones_hbm, onesv)

        @pl.when(s == 0)
        def _zero():
            pltpu.sync_copy(zs_hbm, tbl)
            pltpu.sync_copy(zc_hbm, ctbl)

        plsc.subcore_barrier()

        def cid_of(k_it):
            return gw + NW * k_it

        def stage(k_it, b):
            base = RTC + cid_of(k_it) * CH
            pltpu.async_copy(emb_hbm.at[pl.ds(base, CH)], bufs[b], sem_st[b])
            pltpu.async_copy(lab_hbm.at[pl.ds(base, CH)], idxs[b], sem_st[b])

        def wait_stage(b):
            pltpu.make_async_copy(
                emb_hbm.at[pl.ds(0, CH)], bufs[b], sem_st[b]).wait()
            pltpu.make_async_copy(
                lab_hbm.at[pl.ds(0, CH)], idxs[b], sem_st[b]).wait()

        def fire_scatter(b):
            pltpu.async_copy(bufs[b], tbl.at[idxs[b]], sem_sc[b], add=True)
            pltpu.async_copy(onesv, ctbl.at[idxs[b]], sem_sc[b], add=True)

        def drain_scatter(b):
            pltpu.make_async_copy(bufs[b], tbl.at[idxs[b]], sem_sc[b]).wait()
            pltpu.make_async_copy(onesv, ctbl.at[idxs[b]], sem_sc[b]).wait()

        @pl.when(cid_of(0) < NCHF)
        def _prime():
            stage(0, 0)

        for k_it in range(ROUNDS):
            b = k_it % 2
            if k_it >= 1:
                @pl.when(cid_of(k_it - 1) < NCHF)
                def _drain_prev(b=b):
                    drain_scatter(1 - b)

            @pl.when(cid_of(k_it) < NCHF)
            def _iter(k_it=k_it, b=b):
                if k_it + 1 < ROUNDS:
                    @pl.when(cid_of(k_it + 1) < NCHF)
                    def _next():
                        stage(k_it + 1, 1 - b)
                wait_stage(b)
                fire_scatter(b)

        @pl.when(cid_of(ROUNDS - 1) < NCHF)
        def _drain_last():
            drain_scatter((ROUNDS - 1) % 2)

        @pl.when(gw == NW - 1)
        def _tail():
            base = RTC + NCHF * CH
            pltpu.sync_copy(emb_hbm.at[pl.ds(base, TAIL)], buft)
            pltpu.sync_copy(lab_hbm.at[pl.ds(base, TAIL)], idxt)
            pltpu.async_copy(buft, tbl.at[idxt], sem_sc0, add=True).wait()
            pltpu.async_copy(onesv.at[pl.ds(0, TAIL)], ctbl.at[idxt],
                             sem_sc1, add=True).wait()

        plsc.subcore_barrier()

        @pl.when(s == 0)
        def _flush():
            pltpu.sync_copy(tbl, sums_out.at[c])
            pltpu.sync_copy(ctbl, cnts_out.at[c])

    return k(embeddings, labels_i32, zsums, zcnts, ones_cnt)


def _tc_p1_body(labr_ref, emb_ref, sums_ref, counts_ref):
    g = pl.program_id(0)

    @pl.when(g == 0)
    def _init():
        sums_ref[...] = jnp.zeros_like(sums_ref)
        counts_ref[...] = jnp.zeros_like(counts_ref)

    lab_r = labr_ref[0]
    iota_col = jax.lax.broadcasted_iota(jnp.int32, (KP, 1), 0)
    onehot_kb = (iota_col == lab_r).astype(jnp.float32)
    emb = emb_ref[...]
    sums_ref[...] += jax.lax.dot_general(
        onehot_kb, emb, (((1,), (0,)), ((), ())),
        preferred_element_type=jnp.float32)
    counts_ref[...] += jnp.sum(onehot_kb, axis=1, keepdims=True)


def _tc_partial_sums(embeddings, labi):
    lab_row = labi.reshape(M // B1, 1, B1)
    return pl.pallas_call(
        _tc_p1_body,
        grid=(NB1,),
        in_specs=[
            pl.BlockSpec((1, 1, B1), lambda g: (g, 0, 0)),
            pl.BlockSpec((B1, E), lambda g: (g, 0)),
        ],
        out_specs=[
            pl.BlockSpec((KP, E), lambda g: (0, 0)),
            pl.BlockSpec((KP, 1), lambda g: (0, 0)),
        ],
        out_shape=[
            jax.ShapeDtypeStruct((KP, E), jnp.float32),
            jax.ShapeDtypeStruct((KP, 1), jnp.float32),
        ],
    )(lab_row, embeddings)


def _tc_body(labr_ref, emb_ref, sums_ref, cnts_ref, stc_ref, ctc_ref, out_ref,
             means_ref, msqh_ref, msq_ref, counts_ref, pulls_ref):
    g = pl.program_id(0)

    @pl.when(g == 0)
    def _init():
        sums = sums_ref[0] + sums_ref[1] + stc_ref[...]
        counts = (cnts_ref[0, :, 0:1] + cnts_ref[1, :, 0:1]
                  + ctc_ref[...])
        counts_ref[...] = counts
        safe = jnp.maximum(counts, 1.0)
        means = sums / safe
        means_ref[...] = means
        msq = jnp.sum(means * means, axis=1, keepdims=True)
        msq_ref[...] = msq
        msqh_ref[...] = 0.5 * msq
        pulls_ref[...] = jnp.zeros_like(pulls_ref)

    lab_r = labr_ref[0]
    iota_col = jax.lax.broadcasted_iota(jnp.int32, (KP, 1), 0)
    onehot_kb = (iota_col == lab_r).astype(jnp.float32)
    emb = emb_ref[...]
    dotsT = jax.lax.dot_general(
        means_ref[...], emb, (((1,), (1,)), ((), ())),
        preferred_element_type=jnp.float32)
    e2 = jax.lax.dot_general(
        jnp.ones((1, E), jnp.float32), emb * emb, (((1,), (1,)), ((), ())),
        preferred_element_type=jnp.float32)
    sel = jnp.sum((dotsT - msqh_ref[...]) * onehot_kb, axis=0,
                  keepdims=True)
    d2 = jnp.maximum(e2 - 2.0 * sel, 0.0)
    dist = jnp.sqrt(d2 + 1e-12)
    pull_r = jnp.square(jnp.maximum(dist - DELTA_PULL, 0.0))
    pulls_ref[...] += jnp.sum(onehot_kb * pull_r, axis=1, keepdims=True)

    @pl.when(g == NB - 1)
    def _final():
        counts = counts_ref[...]
        safe = jnp.maximum(counts, 1.0)
        iota_c = jax.lax.broadcasted_iota(jnp.int32, (KP, 1), 0)
        valid = (counts > 0.0) & (iota_c > 0)
        validf = valid.astype(jnp.float32)
        C = jnp.sum(validf)
        Cs = jnp.maximum(C, 1.0)

        ii = jax.lax.broadcasted_iota(jnp.int32, (KP, KP), 0)
        jj = jax.lax.broadcasted_iota(jnp.int32, (KP, KP), 1)
        eye = (ii == jj).astype(jnp.float32)
        safe_row = jnp.sum(eye * safe, axis=0, keepdims=True)
        valid_rowf = jnp.sum(eye * validf, axis=0, keepdims=True)
        msq_col = msq_ref[...]
        msq_row = jnp.sum(eye * msq_col, axis=0, keepdims=True)

        pull_loss = jnp.sum(
            jnp.where(valid, pulls_ref[...] / safe, 0.0)) / Cs

        means = means_ref[...]
        G = jax.lax.dot_general(
            means, means, (((1,), (1,)), ((), ())),
            preferred_element_type=jnp.float32)
        sq = jnp.maximum(msq_col + msq_row - 2.0 * G, 0.0)
        pm = validf * valid_rowf * (ii < jj).astype(jnp.float32)
        d = jnp.sqrt(jnp.where(pm > 0.0, sq, 1.0))
        push = jnp.square(jnp.maximum(2.0 * DELTA_PUSH - d, 0.0))
        n_pairs = jnp.sum(pm)
        push_loss = jnp.where(
            n_pairs > 0.0, jnp.sum(push * pm) / jnp.maximum(n_pairs, 1.0), 0.0)

        mnorm = jnp.sqrt(jnp.where(valid, msq_col, 1.0))
        reg_loss = jnp.sum(jnp.where(valid, mnorm, 0.0)) / Cs

        total = ALPHA * pull_loss + BETA * push_loss + GAMMA * reg_loss
        out_ref[...] = jnp.broadcast_to(total, (1, 1))


@jax.jit
def kernel(embeddings, instance_labels):
    labi = instance_labels.astype(jnp.int32)
    zsums = jnp.zeros((KP, E), jnp.float32)
    zcnts = jnp.zeros((KP, E), jnp.float32)
    ones_cnt = jnp.ones((CH, E), jnp.float32)
    sums2, cnts2 = _sc_segment_sums(embeddings, labi, zsums, zcnts, ones_cnt)
    sums_tc, cnts_tc = _tc_partial_sums(embeddings, labi)

    lab_row = labi.reshape(NB, 1, B)
    out = pl.pallas_call(
        _tc_body,
        grid=(NB,),
        in_specs=[
            pl.BlockSpec((1, 1, B), lambda g: (g, 0, 0)),
            pl.BlockSpec((B, E), lambda g: (g, 0)),
            pl.BlockSpec((NC, KP, E), lambda g: (0, 0, 0)),
            pl.BlockSpec((NC, KP, E), lambda g: (0, 0, 0)),
            pl.BlockSpec((KP, E), lambda g: (0, 0)),
            pl.BlockSpec((KP, 1), lambda g: (0, 0)),
        ],
        out_specs=pl.BlockSpec((1, 1), lambda g: (0, 0)),
        out_shape=jax.ShapeDtypeStruct((1, 1), jnp.float32),
        scratch_shapes=[
            pltpu.VMEM((KP, E), jnp.float32),
            pltpu.VMEM((KP, 1), jnp.float32),
            pltpu.VMEM((KP, 1), jnp.float32),
            pltpu.VMEM((KP, 1), jnp.float32),
            pltpu.VMEM((KP, 1), jnp.float32),
        ],
    )(lab_row, embeddings, sums2, cnts2, sums_tc, cnts_tc)
    return out.reshape(())
